# Initial kernel scaffold; baseline (speedup 1.0000x reference)
import functools

import jax
import jax.numpy as jnp
from jax import lax
from jax.experimental import pallas as pl
from jax.experimental.pallas import tpu as pltpu
from jax.experimental.pallas import tpu_sc as plsc

NUM_CORES = 2
NUM_SUBCORES = 16
NW = NUM_CORES * NUM_SUBCORES

CHUNK = 2048


def _make_gather(B, V, D):
    assert B % NW == 0
    b_per_w = B // NW
    assert b_per_w % CHUNK == 0
    n_chunks = b_per_w // CHUNK

    mesh = plsc.VectorSubcoreMesh(
        core_axis_name="c", subcore_axis_name="s", num_cores=NUM_CORES
    )

    @functools.partial(
        pl.kernel,
        mesh=mesh,
        out_type=jax.ShapeDtypeStruct((B, D), jnp.float32),
        scratch_types=[
            pltpu.VMEM((CHUNK,), jnp.int32),
            pltpu.VMEM((CHUNK, D), jnp.float32),
            pltpu.SemaphoreType.DMA,
        ],
    )
    def gather_kernel(idx_hbm, tbl_hbm, out_hbm, idx_v, rows_v, sem):
        wid = lax.axis_index("s") * NUM_CORES + lax.axis_index("c")
        w_base = wid * b_per_w

        def body(i, carry):
            base = w_base + i * CHUNK
            pltpu.sync_copy(idx_hbm.at[pl.ds(base, CHUNK)], idx_v)
            pltpu.async_copy(tbl_hbm.at[idx_v], rows_v, sem).wait()
            pltpu.sync_copy(rows_v, out_hbm.at[pl.ds(base, CHUNK)])
            return carry

        lax.fori_loop(0, n_chunks, body, 0)

    return gather_kernel


def kernel(x, w):
    s0 = x.shape
    B = x.size
    xf = x.reshape(B)
    out = _make_gather(B, w.shape[0], w.shape[1])(xf, w)
    return out.reshape(*s0, w.shape[1])


# speedup vs baseline: 4.9481x; 4.9481x over previous
import functools

import jax
import jax.numpy as jnp
from jax import lax
from jax.experimental import pallas as pl
from jax.experimental.pallas import tpu as pltpu
from jax.experimental.pallas import tpu_sc as plsc

NUM_CORES = 2
NUM_SUBCORES = 16
NW = NUM_CORES * NUM_SUBCORES

CHUNK = 2048


def _make_gather(B, V, D):
    assert B % NW == 0
    b_per_w = B // NW
    assert b_per_w % CHUNK == 0
    n_chunks = b_per_w // CHUNK

    mesh = plsc.VectorSubcoreMesh(
        core_axis_name="c", subcore_axis_name="s", num_cores=NUM_CORES
    )

    @functools.partial(
        pl.kernel,
        mesh=mesh,
        out_type=jax.ShapeDtypeStruct((B, D), jnp.float32),
        scratch_types=[
            pltpu.VMEM((CHUNK,), jnp.int32),
            pltpu.VMEM((CHUNK, D), jnp.float32),
            pltpu.SemaphoreType.DMA,
        ],
        compiler_params=pltpu.CompilerParams(use_tc_tiling_on_sc=False),
    )
    def gather_kernel(idx_hbm, tbl_hbm, out_hbm, idx_v, rows_v, sem):
        wid = lax.axis_index("s") * NUM_CORES + lax.axis_index("c")
        w_base = wid * b_per_w

        def body(i, carry):
            base = w_base + i * CHUNK
            pltpu.sync_copy(idx_hbm.at[pl.ds(base, CHUNK)], idx_v)
            pltpu.async_copy(tbl_hbm.at[idx_v], rows_v, sem).wait()
            pltpu.sync_copy(rows_v, out_hbm.at[pl.ds(base, CHUNK)])
            return carry

        lax.fori_loop(0, n_chunks, body, 0)

    return gather_kernel


def kernel(x, w):
    s0 = x.shape
    B = x.size
    xf = x.reshape(B)
    out = _make_gather(B, w.shape[0], w.shape[1])(xf, w)
    return out.reshape(*s0, w.shape[1])


# speedup vs baseline: 5.0416x; 1.0189x over previous
import functools

import jax
import jax.numpy as jnp
from jax import lax
from jax.experimental import pallas as pl
from jax.experimental.pallas import tpu as pltpu
from jax.experimental.pallas import tpu_sc as plsc

NUM_CORES = 2
NUM_SUBCORES = 16
NW = NUM_CORES * NUM_SUBCORES

CHUNK = 1600


def _make_gather(B, V, D):
    assert B % NW == 0
    b_per_w = B // NW
    assert b_per_w % (2 * CHUNK) == 0
    n_chunks = b_per_w // CHUNK

    mesh = plsc.VectorSubcoreMesh(
        core_axis_name="c", subcore_axis_name="s", num_cores=NUM_CORES
    )

    @functools.partial(
        pl.kernel,
        mesh=mesh,
        out_type=jax.ShapeDtypeStruct((B, D), jnp.float32),
        scratch_types=[
            pltpu.VMEM((2, CHUNK), jnp.int32),
            pltpu.VMEM((2, CHUNK, D), jnp.float32),
            pltpu.SemaphoreType.DMA((2,)),
            pltpu.SemaphoreType.DMA((2,)),
            pltpu.SemaphoreType.DMA((2,)),
        ],
        compiler_params=pltpu.CompilerParams(use_tc_tiling_on_sc=False),
    )
    def gather_kernel(idx_hbm, tbl_hbm, out_hbm, idx_v, rows_v, isem, gsem, osem):
        wid = lax.axis_index("s") * NUM_CORES + lax.axis_index("c")
        w_base = wid * b_per_w

        def idx_copy(j, b):
            return pltpu.make_async_copy(
                idx_hbm.at[pl.ds(w_base + j * CHUNK, CHUNK)], idx_v.at[b], isem.at[b]
            )

        def gather_copy(b):
            return pltpu.make_async_copy(
                tbl_hbm.at[idx_v.at[b]], rows_v.at[b], gsem.at[b]
            )

        def out_copy(j, b):
            return pltpu.make_async_copy(
                rows_v.at[b], out_hbm.at[pl.ds(w_base + j * CHUNK, CHUNK)], osem.at[b]
            )

        idx_copy(0, 0).start()
        idx_copy(1, 1).start()

        def body(t, carry):
            for b in (0, 1):
                j = 2 * t + b
                idx_copy(j, b).wait()

                @pl.when(j >= 2)
                def _():
                    out_copy(j - 2, b).wait()

                gather_copy(b).start()
                gather_copy(b).wait()

                @pl.when(j + 2 < n_chunks)
                def _():
                    idx_copy(j + 2, b).start()

                out_copy(j, b).start()
            return carry

        lax.fori_loop(0, n_chunks // 2, body, 0)
        out_copy(n_chunks - 2, 0).wait()
        out_copy(n_chunks - 1, 1).wait()

    return gather_kernel


def kernel(x, w):
    s0 = x.shape
    B = x.size
    xf = x.reshape(B)
    out = _make_gather(B, w.shape[0], w.shape[1])(xf, w)
    return out.reshape(*s0, w.shape[1])


# speedup vs baseline: 5.0496x; 1.0016x over previous
import functools

import jax
import jax.numpy as jnp
from jax import lax
from jax.experimental import pallas as pl
from jax.experimental.pallas import tpu as pltpu
from jax.experimental.pallas import tpu_sc as plsc

NUM_CORES = 2
NUM_SUBCORES = 16
NW = NUM_CORES * NUM_SUBCORES

CHUNK = 1600
NSTREAM = 4
SUB = CHUNK // NSTREAM


def _make_gather(B, V, D):
    assert B % NW == 0
    b_per_w = B // NW
    assert b_per_w % (2 * CHUNK) == 0
    n_chunks = b_per_w // CHUNK

    mesh = plsc.VectorSubcoreMesh(
        core_axis_name="c", subcore_axis_name="s", num_cores=NUM_CORES
    )

    @functools.partial(
        pl.kernel,
        mesh=mesh,
        out_type=jax.ShapeDtypeStruct((B, D), jnp.float32),
        scratch_types=[
            pltpu.VMEM((2, CHUNK), jnp.int32),
            pltpu.VMEM((2, CHUNK, D), jnp.float32),
            pltpu.SemaphoreType.DMA((2,)),
            pltpu.SemaphoreType.DMA((2, NSTREAM)),
            pltpu.SemaphoreType.DMA((2,)),
        ],
        compiler_params=pltpu.CompilerParams(use_tc_tiling_on_sc=False),
    )
    def gather_kernel(idx_hbm, tbl_hbm, out_hbm, idx_v, rows_v, isem, gsem, osem):
        wid = lax.axis_index("s") * NUM_CORES + lax.axis_index("c")
        w_base = wid * b_per_w

        def idx_copy(j, b):
            return pltpu.make_async_copy(
                idx_hbm.at[pl.ds(w_base + j * CHUNK, CHUNK)], idx_v.at[b], isem.at[b]
            )

        def gather_copy(b, k):
            return pltpu.make_async_copy(
                tbl_hbm.at[idx_v.at[b, pl.ds(k * SUB, SUB)]],
                rows_v.at[b, pl.ds(k * SUB, SUB)],
                gsem.at[b, k],
            )

        def out_copy(j, b):
            return pltpu.make_async_copy(
                rows_v.at[b], out_hbm.at[pl.ds(w_base + j * CHUNK, CHUNK)], osem.at[b]
            )

        idx_copy(0, 0).start()
        idx_copy(1, 1).start()

        def body(t, carry):
            for b in (0, 1):
                j = 2 * t + b
                idx_copy(j, b).wait()

                @pl.when(j >= 2)
                def _():
                    out_copy(j - 2, b).wait()

                for k in range(NSTREAM):
                    gather_copy(b, k).start()
                for k in range(NSTREAM):
                    gather_copy(b, k).wait()

                @pl.when(j + 2 < n_chunks)
                def _():
                    idx_copy(j + 2, b).start()

                out_copy(j, b).start()
            return carry

        lax.fori_loop(0, n_chunks // 2, body, 0)
        out_copy(n_chunks - 2, 0).wait()
        out_copy(n_chunks - 1, 1).wait()

    return gather_kernel


def kernel(x, w):
    s0 = x.shape
    B = x.size
    xf = x.reshape(B)
    out = _make_gather(B, w.shape[0], w.shape[1])(xf, w)
    return out.reshape(*s0, w.shape[1])
